# Initial kernel scaffold; baseline (speedup 1.0000x reference)
import jax
import jax.numpy as jnp
from jax.experimental import pallas as pl


def kernel(x, edge_index, edge_attr, batch, descriptors, params):
    raise NotImplementedError("write your pallas kernel here")


# speedup vs baseline: 1.0021x; 1.0021x over previous
import functools
import jax
import jax.numpy as jnp
from jax.experimental import pallas as pl
from jax.experimental.pallas import tpu as pltpu

N = 10000
E = 160000
D = 256
HEADS = 4
G = 256
DESC_DIM = 16
NUM_TARGETS = 5

_BN = 1.0 / (1.0 + 1e-5) ** 0.5


def _head_kernel(sump_ref, maxp_ref, cnt_ref, desc_ref,
                 dnW1_ref, dnb1_ref, dnW2_ref, dnb2_ref,
                 fl0W_ref, fl0b_ref, fl1W_ref, fl1b_ref,
                 thW1_ref, thb1_ref, thW2_ref, thb2_ref, thW3_ref, thb3_ref,
                 out_ref):
    sump = sump_ref[...]
    maxp = maxp_ref[...]
    cnt = cnt_ref[...]
    meanp = sump / jnp.maximum(cnt, 1.0)
    g = jnp.concatenate([meanp, maxp, sump, sump], axis=1)
    d = desc_ref[...]
    d = jnp.maximum(d @ dnW1_ref[...] + dnb1_ref[...], 0.0) * _BN
    d = jnp.maximum(d @ dnW2_ref[...] + dnb2_ref[...], 0.0) * _BN
    z = jnp.concatenate([g, d], axis=1)
    z = jnp.maximum(z @ fl0W_ref[...] + fl0b_ref[...], 0.0) * _BN
    z = jnp.maximum(z @ fl1W_ref[...] + fl1b_ref[...], 0.0) * _BN
    outs = []
    for t in range(NUM_TARGETS):
        y = jnp.maximum(z @ thW1_ref[t] + thb1_ref[t], 0.0)
        y = jnp.maximum(y @ thW2_ref[t] + thb2_ref[t], 0.0)
        y = y @ thW3_ref[t] + thb3_ref[t]
        outs.append(y)
    out_ref[...] = jnp.tanh(jnp.concatenate(outs, axis=1))


def _head_stage(sump, maxp, cnt, descriptors, p):
    thW1 = jnp.stack([p[f"th{t}_W1"] for t in range(NUM_TARGETS)])
    thb1 = jnp.stack([p[f"th{t}_b1"] for t in range(NUM_TARGETS)])
    thW2 = jnp.stack([p[f"th{t}_W2"] for t in range(NUM_TARGETS)])
    thb2 = jnp.stack([p[f"th{t}_b2"] for t in range(NUM_TARGETS)])
    thW3 = jnp.stack([p[f"th{t}_W3"] for t in range(NUM_TARGETS)])
    thb3 = jnp.stack([p[f"th{t}_b3"] for t in range(NUM_TARGETS)])
    return pl.pallas_call(
        _head_kernel,
        out_shape=jax.ShapeDtypeStruct((G, NUM_TARGETS), jnp.float32),
    )(sump, maxp, cnt[:, None], descriptors,
      p["dn_W1"], p["dn_b1"], p["dn_W2"], p["dn_b2"],
      p["fl0_W"], p["fl0_b"], p["fl1_W"], p["fl1_b"],
      thW1, thb1, thW2, thb2, thW3, thb3)


def _gcn(h, src, dst, dinv, W, b):
    norm = dinv[src] * dinv[dst]
    hw = h @ W
    out = jax.ops.segment_sum(hw[src] * norm[:, None], dst, num_segments=N)
    return out + b


def _gat(h, src, dst, W, a_s, a_d, b, hd):
    hh = (h @ W).reshape(N, HEADS, hd)
    al_s = jnp.sum(hh * a_s[None], axis=-1)
    al_d = jnp.sum(hh * a_d[None], axis=-1)
    alpha = jax.nn.leaky_relu(al_s[src] + al_d[dst], negative_slope=0.2)
    mx = jax.ops.segment_max(alpha, dst, num_segments=N)
    mx = jnp.where(jnp.isfinite(mx), mx, 0.0)
    ex = jnp.exp(alpha - mx[dst])
    den = jax.ops.segment_sum(ex, dst, num_segments=N)
    coef = ex / (den[dst] + 1e-16)
    out = jax.ops.segment_sum(hh[src] * coef[:, :, None], dst, num_segments=N)
    return out.mean(axis=1) + b


def kernel(x, edge_index, edge_attr, batch, descriptors, params):
    p = params
    loops = jnp.arange(N)
    src = jnp.concatenate([edge_index[0], loops])
    dst = jnp.concatenate([edge_index[1], loops])
    deg = jax.ops.segment_sum(jnp.ones_like(dst, jnp.float32), dst, num_segments=N)
    dinv = jnp.where(deg > 0, deg ** -0.5, 0.0)
    h = x
    for l in range(2):
        x_in = h
        gcn_out = jax.nn.relu(_gcn(h, src, dst, dinv, p[f"gcn{l}_W"], p[f"gcn{l}_b"]))
        gat_out = jax.nn.relu(_gat(h, src, dst, p[f"gat{l}_W"], p[f"gat{l}_as"], p[f"gat{l}_ad"], p[f"gat{l}_b"], D))
        fi = jnp.concatenate([gcn_out, gat_out], axis=1)
        fw = jax.nn.sigmoid(jax.nn.relu(fi @ p["fn_W1"] + p["fn_b1"]) @ p["fn_W2"] + p["fn_b2"])
        fw = jax.nn.sigmoid(fw)
        h = fw[:, :D] * gcn_out + fw[:, D:] * gat_out + x_in
    cnt = jax.ops.segment_sum(jnp.ones((N,), jnp.float32), batch, num_segments=G)
    sump = jax.ops.segment_sum(h, batch, num_segments=G)
    maxp = jax.ops.segment_max(h, batch, num_segments=G)
    maxp = jnp.where(jnp.isfinite(maxp), maxp, 0.0)
    return _head_stage(sump, maxp, cnt, descriptors, params)


# speedup vs baseline: 8.0041x; 7.9873x over previous
import functools
import jax
import jax.numpy as jnp
from jax import lax
from jax.experimental import pallas as pl
from jax.experimental.pallas import tpu as pltpu
from jax.experimental.pallas import tpu_sc as plsc

N = 10000
E = 160000
D = 256
HEADS = 4
G = 256
NUM_TARGETS = 5

NT = 32
NSUB = 16
TILE = 320
NP = 10304
STAGE = 328
DUMP = 320
CAP = 16384
ECHUNK = 4000
DCH = 512
GCH = 64
ACH = 32
TW = 8
GP = 264

R = NP // 8
_BN = 1.0 / (1.0 + 1e-5) ** 0.5

_mesh = plsc.VectorSubcoreMesh(core_axis_name="c", subcore_axis_name="s")
_SC_PARAMS = pltpu.CompilerParams(
    use_tc_tiling_on_sc=False, needs_layout_passes=False)


def _wid():
    return lax.axis_index("c") * NSUB + lax.axis_index("s")


def _iota16():
    return lax.iota(jnp.int32, 16)


def _splat(e):
    return jnp.zeros((16,), jnp.int32) + e


@functools.partial(
    pl.kernel, mesh=_mesh, compiler_params=_SC_PARAMS,
    out_type=[
        jax.ShapeDtypeStruct((NT, 16), jnp.int32),
        jax.ShapeDtypeStruct((NT, CAP), jnp.int32),
        jax.ShapeDtypeStruct((NT, CAP), jnp.int32),
        jax.ShapeDtypeStruct((NT * TILE,), jnp.float32),
    ],
    scratch_types=[
        pltpu.VMEM((ECHUNK,), jnp.int32),
        pltpu.VMEM((ECHUNK,), jnp.int32),
        pltpu.VMEM((CAP,), jnp.int32),
        pltpu.VMEM((CAP,), jnp.int32),
        pltpu.VMEM((336,), jnp.float32),
        pltpu.VMEM((16,), jnp.int32),
    ],
)
def _bucket(src_hbm, dst_hbm, cnt_hbm, srcb_hbm, dstlb_hbm, deg_hbm,
            ebs, ebd, srcb, dstlb, degl, cntv):
    wid = _wid()
    lo = wid * TILE
    nn = jnp.minimum(N - lo, TILE)
    iota = _iota16()
    ones = jnp.ones((16,), jnp.float32)

    for i in range(21):
        degl[pl.ds(i * 16, 16)] = jnp.zeros((16,), jnp.float32)

    def inner(i, ptr):
        sv = ebs[pl.ds(i * 16, 16)]
        dv = ebd[pl.ds(i * 16, 16)]
        m = (dv >= lo) & (dv < lo + TILE)
        cs = plsc.cumsum(m.astype(jnp.int32))
        pos = ptr + cs - 1
        plsc.store_scatter(srcb, [pos], sv, mask=m)
        plsc.store_scatter(dstlb, [pos], dv - lo, mask=m)
        plsc.addupdate_scatter(degl, [dv - lo], ones, mask=m)
        return ptr + plsc.all_reduce_population_count(m)

    def outer(o, ptr):
        pltpu.sync_copy(src_hbm.at[pl.ds(o * ECHUNK, ECHUNK)], ebs)
        pltpu.sync_copy(dst_hbm.at[pl.ds(o * ECHUNK, ECHUNK)], ebd)
        return lax.fori_loop(0, ECHUNK // 16, inner, ptr)

    ptr = lax.fori_loop(0, E // ECHUNK, outer, jnp.zeros((16,), jnp.int32))

    def loops(j, ptr):
        lidx = iota + j * 16
        m = lidx < nn
        cs = plsc.cumsum(m.astype(jnp.int32))
        pos = ptr + cs - 1
        plsc.store_scatter(srcb, [pos], lidx + lo, mask=m)
        plsc.store_scatter(dstlb, [pos], lidx, mask=m)
        plsc.addupdate_scatter(degl, [lidx], ones, mask=m)
        return ptr + plsc.all_reduce_population_count(m)

    ptr = lax.fori_loop(0, TILE // 16, loops, ptr)

    npad = (DCH - lax.rem(ptr, DCH)) % DCH

    def pads(j, _):
        k = iota + j * 16
        m = k < npad
        pos = ptr + k
        plsc.store_scatter(srcb, [pos], jnp.zeros((16,), jnp.int32), mask=m)
        plsc.store_scatter(dstlb, [pos], _splat(DUMP), mask=m)
        return 0

    lax.fori_loop(0, DCH // 16, pads, 0)

    cntv[...] = ptr + npad
    pltpu.sync_copy(cntv, cnt_hbm.at[wid])
    pltpu.sync_copy(srcb, srcb_hbm.at[wid])
    pltpu.sync_copy(dstlb, dstlb_hbm.at[wid])
    pltpu.sync_copy(degl.at[pl.ds(0, TILE)], deg_hbm.at[pl.ds(lo, TILE)])


def _pre_body(h_ref, deg_ref, wg_ref, wa_ref, as_ref, ad_ref,
              hg_ref, ha_ref, gt_ref, dt_ref):
    h = h_ref[...]
    hg_ref[...] = h @ wg_ref[...]
    ha = h @ wa_ref[...]
    ha_ref[...] = ha
    deg = deg_ref[...]
    dinv = jnp.where(deg > 0, lax.rsqrt(jnp.maximum(deg, 1e-30)), 0.0)
    als = []
    ald = []
    for hd in range(HEADS):
        blk = ha[:, hd * D:(hd + 1) * D]
        als.append(jnp.sum(blk * as_ref[hd][None, :], axis=1, keepdims=True))
        ald.append(jnp.sum(blk * ad_ref[hd][None, :], axis=1, keepdims=True))
    z = jnp.zeros((h.shape[0], TW - 5), jnp.float32)
    gt_ref[...] = jnp.concatenate([dinv] + als + [z], axis=1)
    dt_ref[...] = jnp.concatenate([dinv] + ald + [z], axis=1)


def _tc_pre(h, deg, wg, wa, a_s, a_d):
    return pl.pallas_call(
        _pre_body,
        grid=(NP // R,),
        in_specs=[
            pl.BlockSpec((R, D), lambda i: (i, 0)),
            pl.BlockSpec((R, 1), lambda i: (i, 0)),
            pl.BlockSpec((D, D), lambda i: (0, 0)),
            pl.BlockSpec((D, HEADS * D), lambda i: (0, 0)),
            pl.BlockSpec((HEADS, D), lambda i: (0, 0)),
            pl.BlockSpec((HEADS, D), lambda i: (0, 0)),
        ],
        out_specs=[
            pl.BlockSpec((R, D), lambda i: (i, 0)),
            pl.BlockSpec((R, HEADS * D), lambda i: (i, 0)),
            pl.BlockSpec((R, TW), lambda i: (i, 0)),
            pl.BlockSpec((R, TW), lambda i: (i, 0)),
        ],
        out_shape=[
            jax.ShapeDtypeStruct((NP, D), jnp.float32),
            jax.ShapeDtypeStruct((NP, HEADS * D), jnp.float32),
            jax.ShapeDtypeStruct((NP, TW), jnp.float32),
            jax.ShapeDtypeStruct((NP, TW), jnp.float32),
        ],
    )(h, deg, wg, wa, a_s, a_d)


@functools.partial(
    pl.kernel, mesh=_mesh, compiler_params=_SC_PARAMS,
    out_type=jax.ShapeDtypeStruct((NT, STAGE * HEADS), jnp.float32),
    scratch_types=[
        pltpu.VMEM((DCH,), jnp.int32),
        pltpu.VMEM((DCH,), jnp.int32),
        pltpu.VMEM((DCH, TW), jnp.float32),
        pltpu.VMEM((STAGE, TW), jnp.float32),
        pltpu.VMEM((STAGE * HEADS,), jnp.float32),
        pltpu.VMEM((16,), jnp.int32),
        pltpu.SemaphoreType.DMA,
    ],
)
def _den(cnt_hbm, srcb_hbm, dstlb_hbm, gtab_hbm, dtab_hbm, den_hbm,
         srcc, dstc, gbuf, dtl, den, cbuf, sem):
    wid = _wid()
    lo = wid * TILE
    iota = _iota16()
    pltpu.sync_copy(dtab_hbm.at[pl.ds(lo, STAGE)], dtl)
    for i in range(STAGE * HEADS // 16):
        den[pl.ds(i * 16, 16)] = jnp.zeros((16,), jnp.float32)
    pltpu.sync_copy(cnt_hbm.at[wid], cbuf)
    nch = jnp.max(cbuf[...]) // DCH

    def chunk(ci, _):
        off = ci * DCH
        pltpu.sync_copy(srcb_hbm.at[wid, pl.ds(off, DCH)], srcc)
        pltpu.sync_copy(dstlb_hbm.at[wid, pl.ds(off, DCH)], dstc)
        pltpu.async_copy(gtab_hbm.at[srcc], gbuf, sem).wait()
        for j in range(DCH // 16):
            lidx = iota + j * 16
            dl = dstc[pl.ds(j * 16, 16)]
            for hd in range(HEADS):
                col = _splat(hd + 1)
                a = (plsc.load_gather(gbuf, [lidx, col])
                     + plsc.load_gather(dtl, [dl, col]))
                a = jnp.where(a > 0, a, 0.2 * a)
                ex = jnp.exp(a)
                plsc.addupdate_scatter(den, [dl * HEADS + hd], ex)
        return 0

    lax.fori_loop(0, nch, chunk, 0)
    pltpu.sync_copy(den, den_hbm.at[wid])


@functools.partial(
    pl.kernel, mesh=_mesh, compiler_params=_SC_PARAMS,
    out_type=jax.ShapeDtypeStruct((NP, D), jnp.float32),
    scratch_types=[
        pltpu.VMEM((GCH,), jnp.int32),
        pltpu.VMEM((GCH,), jnp.int32),
        pltpu.VMEM((GCH, TW), jnp.float32),
        pltpu.VMEM((GCH, D), jnp.float32),
        pltpu.VMEM((GCH, D), jnp.float32),
        pltpu.VMEM((GCH,), jnp.int32),
        pltpu.VMEM((GCH,), jnp.float32),
        pltpu.VMEM((STAGE, TW), jnp.float32),
        pltpu.VMEM((16,), jnp.int32),
        pltpu.VMEM_SHARED((NSUB * STAGE, D), jnp.float32),
        pltpu.SemaphoreType.DMA,
    ],
)
def _gcn(cnt_hbm, srcb_hbm, dstlb_hbm, gtab_hbm, dtab_hbm, hg_hbm, zer_hbm,
         out_hbm,
         srcc, dstc, gbuf, hgb, msg, idxb, normb, dtl, cbuf, acc, sem):
    wid = _wid()
    s = lax.axis_index("s")
    lo = wid * TILE
    iota = _iota16()
    pltpu.sync_copy(dtab_hbm.at[pl.ds(lo, STAGE)], dtl)
    pltpu.sync_copy(zer_hbm, acc.at[pl.ds(s * STAGE, STAGE)])
    pltpu.sync_copy(cnt_hbm.at[wid], cbuf)
    nch = jnp.max(cbuf[...]) // GCH
    zcol = _splat(0)

    def chunk(ci, _):
        off = ci * GCH
        pltpu.sync_copy(srcb_hbm.at[wid, pl.ds(off, GCH)], srcc)
        pltpu.sync_copy(dstlb_hbm.at[wid, pl.ds(off, GCH)], dstc)
        pltpu.async_copy(hg_hbm.at[srcc], hgb, sem).wait()
        pltpu.async_copy(gtab_hbm.at[srcc], gbuf, sem).wait()
        for j in range(GCH // 16):
            lidx = iota + j * 16
            dl = dstc[pl.ds(j * 16, 16)]
            dsrc = plsc.load_gather(gbuf, [lidx, zcol])
            ddst = plsc.load_gather(dtl, [dl, zcol])
            normb[pl.ds(j * 16, 16)] = dsrc * ddst
            idxb[pl.ds(j * 16, 16)] = dl + s * STAGE

        def edge(e, _):
            se = _splat(e)
            nb = plsc.load_gather(normb, [se])
            for j2 in range(D // 16):
                col = iota + j2 * 16
                hv = plsc.load_gather(hgb, [se, col])
                plsc.store_scatter(msg, [se, col], hv * nb)
            return 0

        lax.fori_loop(0, GCH, edge, 0)
        pltpu.sync_copy(msg, acc.at[idxb], add=True)
        return 0

    lax.fori_loop(0, nch, chunk, 0)
    pltpu.sync_copy(acc.at[pl.ds(s * STAGE, TILE)], out_hbm.at[pl.ds(lo, TILE)])


@functools.partial(
    pl.kernel, mesh=_mesh, compiler_params=_SC_PARAMS,
    out_type=jax.ShapeDtypeStruct((NP, D), jnp.float32),
    scratch_types=[
        pltpu.VMEM((ACH,), jnp.int32),
        pltpu.VMEM((ACH,), jnp.int32),
        pltpu.VMEM((ACH, TW), jnp.float32),
        pltpu.VMEM((ACH, HEADS * D), jnp.float32),
        pltpu.VMEM((ACH, D), jnp.float32),
        pltpu.VMEM((ACH,), jnp.int32),
        pltpu.VMEM((HEADS * ACH,), jnp.float32),
        pltpu.VMEM((STAGE, TW), jnp.float32),
        pltpu.VMEM((STAGE * HEADS,), jnp.float32),
        pltpu.VMEM((16,), jnp.int32),
        pltpu.VMEM_SHARED((NSUB * STAGE, D), jnp.float32),
        pltpu.SemaphoreType.DMA,
    ],
)
def _gat(cnt_hbm, srcb_hbm, dstlb_hbm, gtab_hbm, dtab_hbm, den_hbm, ha_hbm,
         zer_hbm, out_hbm,
         srcc, dstc, gbuf, hab, msg, idxb, coefb, dtl, den, cbuf, acc, sem):
    wid = _wid()
    s = lax.axis_index("s")
    lo = wid * TILE
    iota = _iota16()
    pltpu.sync_copy(dtab_hbm.at[pl.ds(lo, STAGE)], dtl)
    pltpu.sync_copy(den_hbm.at[wid], den)
    pltpu.sync_copy(zer_hbm, acc.at[pl.ds(s * STAGE, STAGE)])
    pltpu.sync_copy(cnt_hbm.at[wid], cbuf)
    nch = jnp.max(cbuf[...]) // ACH

    def chunk(ci, _):
        off = ci * ACH
        pltpu.sync_copy(srcb_hbm.at[wid, pl.ds(off, ACH)], srcc)
        pltpu.sync_copy(dstlb_hbm.at[wid, pl.ds(off, ACH)], dstc)
        pltpu.async_copy(ha_hbm.at[srcc], hab, sem).wait()
        pltpu.async_copy(gtab_hbm.at[srcc], gbuf, sem).wait()
        for j in range(ACH // 16):
            lidx = iota + j * 16
            dl = dstc[pl.ds(j * 16, 16)]
            idxb[pl.ds(j * 16, 16)] = dl + s * STAGE
            for hd in range(HEADS):
                col = _splat(hd + 1)
                a = (plsc.load_gather(gbuf, [lidx, col])
                     + plsc.load_gather(dtl, [dl, col]))
                a = jnp.where(a > 0, a, 0.2 * a)
                ex = jnp.exp(a)
                dg = plsc.load_gather(den, [dl * HEADS + hd])
                coefb[pl.ds(hd * ACH + j * 16, 16)] = (
                    ex / (dg + 1e-16) * (1.0 / HEADS))

        def edge(e, _):
            se = _splat(e)
            cb = [plsc.load_gather(coefb, [_splat(hd * ACH) + e])
                  for hd in range(HEADS)]
            for j2 in range(D // 16):
                col = iota + j2 * 16
                v = cb[0] * plsc.load_gather(hab, [se, col])
                for hd in range(1, HEADS):
                    v = v + cb[hd] * plsc.load_gather(hab, [se, col + hd * D])
                plsc.store_scatter(msg, [se, col], v)
            return 0

        lax.fori_loop(0, ACH, edge, 0)
        pltpu.sync_copy(msg, acc.at[idxb], add=True)
        return 0

    lax.fori_loop(0, nch, chunk, 0)
    pltpu.sync_copy(acc.at[pl.ds(s * STAGE, TILE)], out_hbm.at[pl.ds(lo, TILE)])


def _fuse_body(og_ref, oa_ref, h_ref, bg_ref, ba_ref,
               w1_ref, b1_ref, w2_ref, b2_ref, out_ref):
    gcn = jnp.maximum(og_ref[...] + bg_ref[...], 0.0)
    gat = jnp.maximum(oa_ref[...] + ba_ref[...], 0.0)
    fi = jnp.concatenate([gcn, gat], axis=1)
    t1 = jnp.maximum(fi @ w1_ref[...] + b1_ref[...], 0.0)
    fw = jax.nn.sigmoid(jax.nn.sigmoid(t1 @ w2_ref[...] + b2_ref[...]))
    out_ref[...] = fw[:, :D] * gcn + fw[:, D:] * gat + h_ref[...]


def _tc_fuse(outg, outa, h, bg, ba, w1, b1, w2, b2):
    return pl.pallas_call(
        _fuse_body,
        grid=(NP // R,),
        in_specs=[
            pl.BlockSpec((R, D), lambda i: (i, 0)),
            pl.BlockSpec((R, D), lambda i: (i, 0)),
            pl.BlockSpec((R, D), lambda i: (i, 0)),
            pl.BlockSpec((1, D), lambda i: (0, 0)),
            pl.BlockSpec((1, D), lambda i: (0, 0)),
            pl.BlockSpec((2 * D, D), lambda i: (0, 0)),
            pl.BlockSpec((1, D), lambda i: (0, 0)),
            pl.BlockSpec((D, 2 * D), lambda i: (0, 0)),
            pl.BlockSpec((1, 2 * D), lambda i: (0, 0)),
        ],
        out_specs=pl.BlockSpec((R, D), lambda i: (i, 0)),
        out_shape=jax.ShapeDtypeStruct((NP, D), jnp.float32),
    )(outg, outa, h, bg[None, :], ba[None, :], w1, b1[None, :], w2, b2[None, :])


@functools.partial(
    pl.kernel, mesh=_mesh, compiler_params=_SC_PARAMS,
    out_type=[
        jax.ShapeDtypeStruct((2, GP, D), jnp.float32),
        jax.ShapeDtypeStruct((2, GP, TW), jnp.float32),
        jax.ShapeDtypeStruct((NT, GP, D), jnp.float32),
    ],
    scratch_types=[
        pltpu.VMEM((16, D), jnp.float32),
        pltpu.VMEM((TILE,), jnp.int32),
        pltpu.VMEM((16,), jnp.int32),
        pltpu.VMEM((16, TW), jnp.float32),
        pltpu.VMEM((GP, D), jnp.float32),
        pltpu.VMEM_SHARED((GP, D), jnp.float32),
        pltpu.VMEM_SHARED((GP, TW), jnp.float32),
    ],
)
def _pool(h_hbm, batch_hbm, zsum_hbm, zcnt_hbm, ninf_hbm, ones_hbm,
          sum_hbm, cnt_hbm, max_hbm,
          hbuf, bbuf, idx16, obuf, pmax, accs, accc):
    wid = _wid()
    c = lax.axis_index("c")
    s = lax.axis_index("s")
    lo = wid * TILE
    iota = _iota16()

    @pl.when(s == 0)
    def _():
        pltpu.sync_copy(zsum_hbm, accs)
        pltpu.sync_copy(zcnt_hbm, accc)

    pltpu.sync_copy(ninf_hbm, pmax)
    pltpu.sync_copy(ones_hbm, obuf)
    pltpu.sync_copy(batch_hbm.at[pl.ds(lo, TILE)], bbuf)
    plsc.subcore_barrier()

    def node16(i, _):
        base = i * 16
        pltpu.sync_copy(h_hbm.at[pl.ds(lo + base, 16)], hbuf)
        gv = bbuf[pl.ds(base, 16)]
        idx16[...] = gv
        pltpu.sync_copy(hbuf, accs.at[idx16], add=True)
        pltpu.sync_copy(obuf, accc.at[idx16], add=True)

        def node(e, _):
            se = _splat(e)
            g = plsc.load_gather(bbuf, [_splat(base) + e])
            for j2 in range(D // 16):
                col = iota + j2 * 16
                hv = plsc.load_gather(hbuf, [se, col])
                pv = plsc.load_gather(pmax, [g, col])
                plsc.store_scatter(pmax, [g, col], jnp.maximum(pv, hv))
            return 0

        lax.fori_loop(0, 16, node, 0)
        return 0

    lax.fori_loop(0, TILE // 16, node16, 0)
    plsc.subcore_barrier()
    pltpu.sync_copy(accs.at[pl.ds(s * 16, 16)], sum_hbm.at[c, pl.ds(s * 16, 16)])
    pltpu.sync_copy(accc.at[pl.ds(s * 16, 16)], cnt_hbm.at[c, pl.ds(s * 16, 16)])

    @pl.when(s == 0)
    def _():
        pltpu.sync_copy(accs.at[pl.ds(G, GP - G)], sum_hbm.at[c, pl.ds(G, GP - G)])
        pltpu.sync_copy(accc.at[pl.ds(G, GP - G)], cnt_hbm.at[c, pl.ds(G, GP - G)])

    pltpu.sync_copy(pmax, max_hbm.at[wid])


def _head_body(sp_ref, cp_ref, mp_ref, desc_ref,
               dnW1_ref, dnb1_ref, dnW2_ref, dnb2_ref,
               fl0W_ref, fl0b_ref, fl1W_ref, fl1b_ref,
               thW1_ref, thb1_ref, thW2_ref, thb2_ref, thW3_ref, thb3_ref,
               out_ref):
    sp = sp_ref[...]
    sump = (sp[0] + sp[1])[:G]
    cnt = (cp_ref[...][0] + cp_ref[...][1])[:G, 0:1]
    maxp = jnp.max(mp_ref[...], axis=0)[:G]
    maxp = jnp.where(maxp == -jnp.inf, 0.0, maxp)
    meanp = sump / jnp.maximum(cnt, 1.0)
    g = jnp.concatenate([meanp, maxp, sump, sump], axis=1)
    d = desc_ref[...]
    d = jnp.maximum(d @ dnW1_ref[...] + dnb1_ref[...], 0.0) * _BN
    d = jnp.maximum(d @ dnW2_ref[...] + dnb2_ref[...], 0.0) * _BN
    z = jnp.concatenate([g, d], axis=1)
    z = jnp.maximum(z @ fl0W_ref[...] + fl0b_ref[...], 0.0) * _BN
    z = jnp.maximum(z @ fl1W_ref[...] + fl1b_ref[...], 0.0) * _BN
    outs = []
    for t in range(NUM_TARGETS):
        y = jnp.maximum(z @ thW1_ref[t] + thb1_ref[t], 0.0)
        y = jnp.maximum(y @ thW2_ref[t] + thb2_ref[t], 0.0)
        y = y @ thW3_ref[t] + thb3_ref[t]
        outs.append(y)
    out_ref[...] = jnp.tanh(jnp.concatenate(outs, axis=1))


def _head(sum_parts, cnt_parts, max_parts, descriptors, p):
    thW1 = jnp.stack([p[f"th{t}_W1"] for t in range(NUM_TARGETS)])
    thb1 = jnp.stack([p[f"th{t}_b1"] for t in range(NUM_TARGETS)])
    thW2 = jnp.stack([p[f"th{t}_W2"] for t in range(NUM_TARGETS)])
    thb2 = jnp.stack([p[f"th{t}_b2"] for t in range(NUM_TARGETS)])
    thW3 = jnp.stack([p[f"th{t}_W3"] for t in range(NUM_TARGETS)])
    thb3 = jnp.stack([p[f"th{t}_b3"] for t in range(NUM_TARGETS)])
    return pl.pallas_call(
        _head_body,
        out_shape=jax.ShapeDtypeStruct((G, NUM_TARGETS), jnp.float32),
    )(sum_parts, cnt_parts, max_parts, descriptors,
      p["dn_W1"], p["dn_b1"], p["dn_W2"], p["dn_b2"],
      p["fl0_W"], p["fl0_b"], p["fl1_W"], p["fl1_b"],
      thW1, thb1, thW2, thb2, thW3, thb3)


def kernel(x, edge_index, edge_attr, batch, descriptors, params):
    p = params
    src = edge_index[0].astype(jnp.int32)
    dst = edge_index[1].astype(jnp.int32)

    cnt32, srcb, dstlb, deg = _bucket(src, dst)
    deg_pad = jnp.pad(deg, (0, NP - NT * TILE))[:, None]

    h = jnp.pad(x, ((0, NP - N), (0, 0)))
    zer = jnp.zeros((STAGE, D), jnp.float32)

    for l in range(2):
        hg, ha, gtab, dtab = _tc_pre(
            h, deg_pad, p[f"gcn{l}_W"], p[f"gat{l}_W"],
            p[f"gat{l}_as"], p[f"gat{l}_ad"])
        den = _den(cnt32, srcb, dstlb, gtab, dtab)
        outg = _gcn(cnt32, srcb, dstlb, gtab, dtab, hg, zer)
        outa = _gat(cnt32, srcb, dstlb, gtab, dtab, den, ha, zer)
        h = _tc_fuse(outg, outa, h, p[f"gcn{l}_b"], p[f"gat{l}_b"],
                     p["fn_W1"], p["fn_b1"], p["fn_W2"], p["fn_b2"])

    batch_pad = jnp.pad(batch.astype(jnp.int32), (0, NP - N),
                        constant_values=G)
    zsum = jnp.zeros((GP, D), jnp.float32)
    zcnt = jnp.zeros((GP, TW), jnp.float32)
    ninf = jnp.full((GP, D), -jnp.inf, jnp.float32)
    ones16 = jnp.ones((16, TW), jnp.float32)
    sum_parts, cnt_parts, max_parts = _pool(
        h, batch_pad, zsum, zcnt, ninf, ones16)
    return _head(sum_parts, cnt_parts, max_parts, descriptors, params)
